# Initial kernel scaffold; baseline (speedup 1.0000x reference)
import jax
import jax.numpy as jnp
from jax.experimental import pallas as pl


def kernel(x, emb):
    raise NotImplementedError("write your pallas kernel here")


# speedup vs baseline: 1.6661x; 1.6661x over previous
import jax
import jax.numpy as jnp
from jax.experimental import pallas as pl

BT = 1024


def _add_kernel(x_ref, emb_ref, o_ref):
    o_ref[...] = x_ref[...] + emb_ref[...]


def kernel(x, emb):
    B, T, E = x.shape
    pe = emb[:T]
    grid = (T // BT, B)
    return pl.pallas_call(
        _add_kernel,
        grid=grid,
        in_specs=[
            pl.BlockSpec((1, BT, E), lambda t, b: (b, t, 0)),
            pl.BlockSpec((BT, E), lambda t, b: (t, 0)),
        ],
        out_specs=pl.BlockSpec((1, BT, E), lambda t, b: (b, t, 0)),
        out_shape=jax.ShapeDtypeStruct((B, T, E), x.dtype),
    )(x, pe)


# speedup vs baseline: 1.7349x; 1.0413x over previous
import jax
import jax.numpy as jnp
from jax.experimental import pallas as pl

BT = 2048


def _add_kernel(x_ref, emb_ref, o_ref):
    o_ref[...] = x_ref[...] + emb_ref[...]


def kernel(x, emb):
    B, T, E = x.shape
    pe = emb[:T]
    grid = (T // BT, B)
    return pl.pallas_call(
        _add_kernel,
        grid=grid,
        in_specs=[
            pl.BlockSpec((1, BT, E), lambda t, b: (b, t, 0)),
            pl.BlockSpec((BT, E), lambda t, b: (t, 0)),
        ],
        out_specs=pl.BlockSpec((1, BT, E), lambda t, b: (b, t, 0)),
        out_shape=jax.ShapeDtypeStruct((B, T, E), x.dtype),
    )(x, pe)
